# baseline (speedup 1.0000x reference)
import functools
import math

import jax
import jax.numpy as jnp
from jax import lax
from jax.experimental import pallas as pl
from jax.experimental.pallas import tpu as pltpu
from jax.experimental.pallas import tpu_sc as plsc

NC = 2
NS = 16
NW = NC * NS
LANES = 16
CHUNK = 128
NBUF = 4
PADV = CHUNK


def _emb_kernel(S, V, D):
  scale = math.sqrt(D)
  mesh = plsc.VectorSubcoreMesh(core_axis_name="c", subcore_axis_name="s")
  DI = D // 8
  ST = S // 8
  NFB = D // LANES

  @functools.partial(
      pl.kernel,
      mesh=mesh,
      compiler_params=pltpu.CompilerParams(
          use_tc_tiling_on_sc=False, needs_layout_passes=False),
      out_type=jax.ShapeDtypeStruct((S, DI, NW, 8, CHUNK), jnp.float32),
      scratch_types=[
          pltpu.VMEM((ST, 8, CHUNK), jnp.int32),
          pltpu.VMEM((NBUF, CHUNK, D), jnp.float32),
          [pltpu.VMEM((NBUF, 2, 8, PADV), jnp.float32)] * NFB,
          [pltpu.SemaphoreType.DMA] * NBUF,
          [pltpu.SemaphoreType.DMA] * NBUF,
      ],
  )
  def k(idx_hbm, table_hbm, out_hbm, idx_v, rbuf, tbufs, gsem, ssem):
    wid = lax.axis_index("s") * NC + lax.axis_index("c")
    pltpu.sync_copy(idx_hbm.at[:, wid], idx_v)
    lanes = lax.iota(jnp.int32, LANES)
    fcoords = []
    for fb in range(NFB):
      f16 = fb * LANES + lanes
      fcoords.append((f16 // 8 - 2 * fb, f16 % 8))

    for b in range(NBUF):
      pltpu.async_copy(table_hbm.at[idx_v.at[b // 8, b % 8]],
                       rbuf.at[b], gsem[b])

    @pl.loop(0, S, step=NBUF)
    def _grp(s0):
      for b in range(NBUF):
        s = s0 + b
        pltpu.make_async_copy(
            table_hbm.at[idx_v.at[0, 0]], rbuf.at[b], gsem[b]).wait()

        rb = rbuf.at[b]
        tbs = [t.at[b] for t in tbufs]

        @pl.when(s >= NBUF)
        def _():
          for tb in tbs:
            pltpu.make_async_copy(
                tb.at[:, :, pl.ds(0, CHUNK)],
                out_hbm.at[0, pl.ds(0, 2), wid], ssem[b]).wait()

        @pl.loop(0, CHUNK, unroll=4)
        def _v(v):
          vcol = jnp.full((LANES,), v, jnp.int32)
          for fb, (fi, fu) in enumerate(fcoords):
            vals = rb[v, pl.ds(fb * LANES, LANES)]
            plsc.store_scatter(tbs[fb], [fi, fu, vcol], vals * scale)

        @pl.when(s + NBUF < S)
        def _():
          sn = s + NBUF
          pltpu.async_copy(
              table_hbm.at[idx_v.at[sn // 8, sn % 8]], rbuf.at[b], gsem[b])

        for fb, tb in enumerate(tbs):
          pltpu.async_copy(
              tb.at[:, :, pl.ds(0, CHUNK)],
              out_hbm.at[s, pl.ds(2 * fb, 2), wid], ssem[b])

    for b in range(NBUF):
      for t in tbufs:
        pltpu.make_async_copy(
            t.at[b].at[:, :, pl.ds(0, CHUNK)],
            out_hbm.at[0, pl.ds(0, 2), wid], ssem[b]).wait()

  return k


def kernel(tokens, table):
  B0, S = tokens.shape
  V, D = table.shape
  idx = jnp.clip(tokens.astype(jnp.int32), 0, V - 1)
  idx = idx.T.reshape(S // 8, 8, NW, CHUNK).transpose(0, 2, 1, 3)
  out5 = _emb_kernel(S, V, D)(idx, table)
  return out5.transpose(2, 4, 0, 1, 3).reshape(B0, S, D)


# speedup vs baseline: 2.2865x; 2.2865x over previous
import functools
import math

import jax
import jax.numpy as jnp
from jax import lax
from jax.experimental import pallas as pl
from jax.experimental.pallas import tpu as pltpu
from jax.experimental.pallas import tpu_sc as plsc

NC = 2
NS = 16
NW = NC * NS
LANES = 16
CHUNK = 128
NBUF = 4
PADV = CHUNK + 8


def _emb_kernel(S, V, D):
  scale = math.sqrt(D)
  mesh = plsc.VectorSubcoreMesh(core_axis_name="c", subcore_axis_name="s")
  DI = D // 8
  ST = S // 8

  @functools.partial(
      pl.kernel,
      mesh=mesh,
      compiler_params=pltpu.CompilerParams(
          use_tc_tiling_on_sc=False, needs_layout_passes=False),
      out_type=jax.ShapeDtypeStruct((S, DI, NW, 8, CHUNK), jnp.float32),
      scratch_types=[
          pltpu.VMEM((ST, 8, CHUNK), jnp.int32),
          pltpu.VMEM((NBUF, CHUNK, D), jnp.float32),
          pltpu.VMEM((NBUF, DI, 8, PADV), jnp.float32),
          [pltpu.SemaphoreType.DMA] * NBUF,
          [pltpu.SemaphoreType.DMA] * NBUF,
      ],
  )
  def k(idx_hbm, table_hbm, out_hbm, idx_v, rbuf, tbuf, gsem, ssem):
    wid = lax.axis_index("s") * NC + lax.axis_index("c")
    pltpu.sync_copy(idx_hbm.at[:, wid], idx_v)
    lanes = lax.iota(jnp.int32, LANES)
    fcoords = []
    for fb in range(D // LANES):
      f16 = fb * LANES + lanes
      fcoords.append((f16 // 8, f16 % 8))

    for b in range(NBUF):
      pltpu.async_copy(table_hbm.at[idx_v.at[b // 8, b % 8]],
                       rbuf.at[b], gsem[b])

    @pl.loop(0, S, step=NBUF)
    def _grp(s0):
      for b in range(NBUF):
        s = s0 + b
        pltpu.make_async_copy(
            table_hbm.at[idx_v.at[0, 0]], rbuf.at[b], gsem[b]).wait()

        rb = rbuf.at[b]
        tb = tbuf.at[b]

        @pl.when(s >= NBUF)
        def _():
          pltpu.make_async_copy(
              tb.at[:, :, pl.ds(0, CHUNK)],
              out_hbm.at[0, :, wid], ssem[b]).wait()

        @plsc.parallel_loop(0, CHUNK, unroll=4)
        def _v(v):
          vcol = jnp.full((LANES,), v, jnp.int32)
          for fb, (fi, fu) in enumerate(fcoords):
            vals = rb[v, pl.ds(fb * LANES, LANES)]
            plsc.store_scatter(tb, [fi, fu, vcol], vals * scale)

        @pl.when(s + NBUF < S)
        def _():
          sn = s + NBUF
          pltpu.async_copy(
              table_hbm.at[idx_v.at[sn // 8, sn % 8]], rbuf.at[b], gsem[b])

        pltpu.async_copy(
            tb.at[:, :, pl.ds(0, CHUNK)], out_hbm.at[s, :, wid], ssem[b])

    for b in range(NBUF):
      pltpu.make_async_copy(
          tbuf.at[b].at[:, :, pl.ds(0, CHUNK)],
          out_hbm.at[0, :, wid], ssem[b]).wait()

  return k


def kernel(tokens, table):
  B0, S = tokens.shape
  V, D = table.shape
  idx = jnp.clip(tokens.astype(jnp.int32), 0, V - 1)
  idx = idx.T.reshape(S // 8, 8, NW, CHUNK).transpose(0, 2, 1, 3)
  out5 = _emb_kernel(S, V, D)(idx, table)
  return out5.transpose(2, 4, 0, 1, 3).reshape(B0, S, D)
